# baseline (speedup 1.0000x reference)
import functools

import jax
import jax.numpy as jnp
from jax import lax
from jax.experimental import pallas as pl
from jax.experimental.pallas import tpu as pltpu
from jax.experimental.pallas import tpu_sc as plsc

N_CLASSES = 1_000_000
EMBED = 16
BATCH = 16384

_NC = 2
_NS = 16
_NW = _NC * _NS
_BPW = BATCH // _NW
_CHUNK = 128
_NCHUNK = _BPW // _CHUNK

_mesh = plsc.VectorSubcoreMesh(core_axis_name="c", subcore_axis_name="s")


@functools.partial(
    pl.kernel,
    mesh=_mesh,
    out_type=jax.ShapeDtypeStruct((BATCH, EMBED), jnp.float32),
    scratch_types=(
        [pltpu.VMEM((_CHUNK,), jnp.int32) for _ in range(_NCHUNK)]
        + [pltpu.VMEM((_BPW, EMBED), jnp.float32), pltpu.SemaphoreType.DMA]
    ),
    compiler_params=pltpu.CompilerParams(use_tc_tiling_on_sc=False),
)
def _gather_kernel(table_hbm, labels_hbm, out_hbm, i0, i1, i2, i3, rows, sem):
    wid = lax.axis_index("s") * _NC + lax.axis_index("c")
    base = wid * _BPW
    idx_bufs = (i0, i1, i2, i3)
    for j in range(_NCHUNK):
        pltpu.sync_copy(labels_hbm.at[pl.ds(base + j * _CHUNK, _CHUNK)],
                        idx_bufs[j])
    copies = [
        pltpu.async_copy(table_hbm.at[idx_bufs[j]],
                         rows.at[pl.ds(j * _CHUNK, _CHUNK)], sem)
        for j in range(_NCHUNK)
    ]
    for c in copies:
        c.wait()
    pltpu.sync_copy(rows, out_hbm.at[pl.ds(base, _BPW)])


def kernel(labels, embed_table):
    return _gather_kernel(embed_table, labels.astype(jnp.int32))
